# baseline (speedup 1.0000x reference)
import functools
import jax
import jax.numpy as jnp
from jax import lax
from jax.experimental import pallas as pl
from jax.experimental.pallas import tpu as pltpu
from jax.experimental.pallas import tpu_sc as plsc

D = 256
NTOPIC = 20
NLAYERS = 3
NCHUNK = 8
CW = 32

NODE_TYPES = ("gene", "peak", "protein")
EDGE_DEFS = (("g2p", "gene", "peak"), ("g2pr", "gene", "protein"),
             ("p2pr", "peak", "protein"), ("pr2pr", "protein", "protein"))
IN_EDGES = {"gene": (), "peak": ("g2p",), "protein": ("g2pr", "p2pr", "pr2pr")}
OUT_EDGES = {"gene": ("g2p", "g2pr"), "peak": ("p2pr",), "protein": ("pr2pr",)}


def _row_block(n):
    if n % 1000 == 0:
        return 1000
    return n


def _proj_kernel(n_edge, x_ref, w_ref, self_ref, *edge_refs):
    acc = jnp.dot(x_ref[...], w_ref[...], preferred_element_type=jnp.float32)
    self_ref[...] = acc[:, :D]
    for j in range(n_edge):
        base = D * (1 + j)
        for c in range(NCHUNK):
            edge_refs[j][c] = acc[:, base + c * CW: base + (c + 1) * CW]


def _project(x, w_self, w_edges):
    n = x.shape[0]
    bn = _row_block(n)
    w = jnp.concatenate([w_self] + list(w_edges), axis=1)
    kout = w.shape[1]
    n_edge = len(w_edges)
    out_shapes = [jax.ShapeDtypeStruct((n, D), jnp.float32)] + [
        jax.ShapeDtypeStruct((NCHUNK, n, CW), jnp.float32) for _ in range(n_edge)
    ]
    out_specs = [pl.BlockSpec((bn, D), lambda i: (i, 0))] + [
        pl.BlockSpec((NCHUNK, bn, CW), lambda i: (0, i, 0)) for _ in range(n_edge)
    ]
    return pl.pallas_call(
        functools.partial(_proj_kernel, n_edge),
        grid=(n // bn,),
        in_specs=[
            pl.BlockSpec((bn, D), lambda i: (i, 0)),
            pl.BlockSpec((D, kout), lambda i: (0, 0)),
        ],
        out_specs=out_specs,
        out_shape=out_shapes,
    )(x, w)


def _combine_kernel(n_edge, self_ref, h_ref, *rest):
    out_ref = rest[-1]
    total = self_ref[...]
    for j in range(n_edge):
        agg_ref = rest[2 * j]
        cnt_ref = rest[2 * j + 1]
        agg = jnp.concatenate([agg_ref[c] for c in range(NCHUNK)], axis=1)
        cnt = jnp.maximum(cnt_ref[:, 0:1], 1.0)
        total = total + agg / cnt
    out_ref[...] = jnp.maximum(total, 0.0) + h_ref[...]


def _combine(selfp, h, aggs, cnts):
    n = h.shape[0]
    bn = _row_block(n)
    n_edge = len(aggs)
    in_specs = [pl.BlockSpec((bn, D), lambda i: (i, 0)),
                pl.BlockSpec((bn, D), lambda i: (i, 0))]
    args = [selfp, h]
    for agg, cnt in zip(aggs, cnts):
        in_specs.append(pl.BlockSpec((NCHUNK, bn, CW), lambda i: (0, i, 0)))
        in_specs.append(pl.BlockSpec((bn, 16), lambda i: (i, 0)))
        args.append(agg)
        args.append(cnt)
    return pl.pallas_call(
        functools.partial(_combine_kernel, n_edge),
        grid=(n // bn,),
        in_specs=in_specs,
        out_specs=pl.BlockSpec((bn, D), lambda i: (i, 0)),
        out_shape=jax.ShapeDtypeStruct((n, D), jnp.float32),
    )(*args)


def _encoder_kernel(x_ref, w1_ref, b1_ref, w2_ref, b2_ref, w3_ref, b3_ref,
                    out_ref):
    h1 = jnp.maximum(
        jnp.dot(x_ref[...], w1_ref[...], preferred_element_type=jnp.float32)
        + b1_ref[...], 0.0)
    h2 = jnp.maximum(
        jnp.dot(h1, w2_ref[...], preferred_element_type=jnp.float32)
        + b2_ref[...], 0.0)
    mu = jnp.dot(h2, w3_ref[...], preferred_element_type=jnp.float32) + b3_ref[...]
    mu = mu - jnp.max(mu, axis=-1, keepdims=True)
    e = jnp.exp(mu)
    out_ref[...] = e / jnp.sum(e, axis=-1, keepdims=True)


def _encode(x, w1, b1, w2, b2, w3, b3):
    n = x.shape[0]
    bn = _row_block(n)
    h1 = w1.shape[1]
    h2 = w2.shape[1]
    return pl.pallas_call(
        _encoder_kernel,
        grid=(n // bn,),
        in_specs=[
            pl.BlockSpec((bn, D), lambda i: (i, 0)),
            pl.BlockSpec((D, h1), lambda i: (0, 0)),
            pl.BlockSpec((1, h1), lambda i: (0, 0)),
            pl.BlockSpec((h1, h2), lambda i: (0, 0)),
            pl.BlockSpec((1, h2), lambda i: (0, 0)),
            pl.BlockSpec((h2, NTOPIC), lambda i: (0, 0)),
            pl.BlockSpec((1, NTOPIC), lambda i: (0, 0)),
        ],
        out_specs=pl.BlockSpec((bn, NTOPIC), lambda i: (i, 0)),
        out_shape=jax.ShapeDtypeStruct((n, NTOPIC), jnp.float32),
    )(x, w1, b1.reshape(1, -1), w2, b2.reshape(1, -1), w3, b3.reshape(1, -1))


def _aggregate_placeholder(proj_chunked, src, dst, n_dst):
    n_src = proj_chunked.shape[1]
    proj = proj_chunked.transpose(1, 0, 2).reshape(n_src, D)
    msgs = jnp.take(proj, src, axis=0)
    s = jax.ops.segment_sum(msgs, dst, num_segments=n_dst)
    return s.reshape(n_dst, NCHUNK, CW).transpose(1, 0, 2)


def _counts_placeholder(dst, n_dst):
    cnt = jax.ops.segment_sum(jnp.ones(dst.shape, jnp.float32), dst,
                              num_segments=n_dst)
    return jnp.tile(cnt[:, None], (1, 16))


def kernel(gene_x, peak_x, protein_x, params, edge_g2p, edge_g2pr, edge_p2pr,
           edge_pr2pr):
    h = {"gene": gene_x, "peak": peak_x, "protein": protein_x}
    sizes = {nt: h[nt].shape[0] for nt in NODE_TYPES}
    edges = {"g2p": edge_g2p, "g2pr": edge_g2pr, "p2pr": edge_p2pr,
             "pr2pr": edge_pr2pr}

    cnts = {}
    for name, _, d_t in EDGE_DEFS:
        cnts[name] = _counts_placeholder(edges[name][1], sizes[d_t])

    for l in range(NLAYERS):
        projs = {}
        selfs = {}
        for nt in NODE_TYPES:
            w_edges = [params["W_%s_%d" % (name, l)] for name in OUT_EDGES[nt]]
            outs = _project(h[nt], params["Wself_%s_%d" % (nt, l)], w_edges)
            selfs[nt] = outs[0]
            for j, name in enumerate(OUT_EDGES[nt]):
                projs[name] = outs[1 + j]

        aggs = {}
        for name, s_t, d_t in EDGE_DEFS:
            aggs[name] = _aggregate_placeholder(
                projs[name], edges[name][0], edges[name][1], sizes[d_t])

        new_h = {}
        for nt in NODE_TYPES:
            names = IN_EDGES[nt]
            new_h[nt] = _combine(selfs[nt], h[nt],
                                 [aggs[m] for m in names],
                                 [cnts[m] for m in names])
        h = new_h

    outs = []
    for nt in NODE_TYPES:
        p = params
        outs.append(_encode(h[nt], p["enc_%s_W1" % nt], p["enc_%s_b1" % nt],
                            p["enc_%s_W2" % nt], p["enc_%s_b2" % nt],
                            p["enc_%s_W3" % nt], p["enc_%s_b3" % nt]))
    return tuple(outs)


# speedup vs baseline: 2.1627x; 2.1627x over previous
import functools
import jax
import jax.numpy as jnp
from jax import lax
from jax.experimental import pallas as pl
from jax.experimental.pallas import tpu as pltpu
from jax.experimental.pallas import tpu_sc as plsc

D = 256
NTOPIC = 20
NLAYERS = 3
NCHUNK = 8
CW = 32

NODE_TYPES = ("gene", "peak", "protein")
EDGE_DEFS = (("g2p", "gene", "peak"), ("g2pr", "gene", "protein"),
             ("p2pr", "peak", "protein"), ("pr2pr", "protein", "protein"))
IN_EDGES = {"gene": (), "peak": ("g2p",), "protein": ("g2pr", "p2pr", "pr2pr")}
OUT_EDGES = {"gene": ("g2p", "g2pr"), "peak": ("p2pr",), "protein": ("pr2pr",)}


def _row_block(n):
    if n % 1000 == 0:
        return 1000
    return n


def _proj_kernel(n_edge, x_ref, w_ref, self_ref, *edge_refs):
    acc = jnp.dot(x_ref[...], w_ref[...], preferred_element_type=jnp.float32)
    self_ref[...] = acc[:, :D]
    for j in range(n_edge):
        base = D * (1 + j)
        for c in range(NCHUNK):
            edge_refs[j][c] = acc[:, base + c * CW: base + (c + 1) * CW]


def _project(x, w_self, w_edges):
    n = x.shape[0]
    bn = _row_block(n)
    w = jnp.concatenate([w_self] + list(w_edges), axis=1)
    kout = w.shape[1]
    n_edge = len(w_edges)
    out_shapes = [jax.ShapeDtypeStruct((n, D), jnp.float32)] + [
        jax.ShapeDtypeStruct((NCHUNK, n, CW), jnp.float32) for _ in range(n_edge)
    ]
    out_specs = [pl.BlockSpec((bn, D), lambda i: (i, 0))] + [
        pl.BlockSpec((NCHUNK, bn, CW), lambda i: (0, i, 0)) for _ in range(n_edge)
    ]
    return pl.pallas_call(
        functools.partial(_proj_kernel, n_edge),
        grid=(n // bn,),
        in_specs=[
            pl.BlockSpec((bn, D), lambda i: (i, 0)),
            pl.BlockSpec((D, kout), lambda i: (0, 0)),
        ],
        out_specs=out_specs,
        out_shape=out_shapes,
    )(x, w)


def _combine_kernel(n_edge, self_ref, h_ref, *rest):
    out_ref = rest[-1]
    total = self_ref[...]
    for j in range(n_edge):
        agg_ref = rest[2 * j]
        cnt_ref = rest[2 * j + 1]
        agg = jnp.concatenate([agg_ref[c] for c in range(NCHUNK)], axis=1)
        cnt = jnp.maximum(cnt_ref[:, 0:1], 1.0)
        total = total + agg / cnt
    out_ref[...] = jnp.maximum(total, 0.0) + h_ref[...]


def _combine(selfp, h, aggs, cnts):
    n = h.shape[0]
    bn = _row_block(n)
    n_edge = len(aggs)
    in_specs = [pl.BlockSpec((bn, D), lambda i: (i, 0)),
                pl.BlockSpec((bn, D), lambda i: (i, 0))]
    args = [selfp, h]
    for agg, cnt in zip(aggs, cnts):
        in_specs.append(pl.BlockSpec((NCHUNK, bn, CW), lambda i: (0, i, 0)))
        in_specs.append(pl.BlockSpec((bn, CW), lambda i: (i, 0)))
        args.append(agg)
        args.append(cnt)
    return pl.pallas_call(
        functools.partial(_combine_kernel, n_edge),
        grid=(n // bn,),
        in_specs=in_specs,
        out_specs=pl.BlockSpec((bn, D), lambda i: (i, 0)),
        out_shape=jax.ShapeDtypeStruct((n, D), jnp.float32),
    )(*args)


def _encoder_kernel(x_ref, w1_ref, b1_ref, w2_ref, b2_ref, w3_ref, b3_ref,
                    out_ref):
    h1 = jnp.maximum(
        jnp.dot(x_ref[...], w1_ref[...], preferred_element_type=jnp.float32)
        + b1_ref[...], 0.0)
    h2 = jnp.maximum(
        jnp.dot(h1, w2_ref[...], preferred_element_type=jnp.float32)
        + b2_ref[...], 0.0)
    mu = jnp.dot(h2, w3_ref[...], preferred_element_type=jnp.float32) + b3_ref[...]
    mu = mu - jnp.max(mu, axis=-1, keepdims=True)
    e = jnp.exp(mu)
    out_ref[...] = e / jnp.sum(e, axis=-1, keepdims=True)


def _encode(x, w1, b1, w2, b2, w3, b3):
    n = x.shape[0]
    bn = _row_block(n)
    h1 = w1.shape[1]
    h2 = w2.shape[1]
    return pl.pallas_call(
        _encoder_kernel,
        grid=(n // bn,),
        in_specs=[
            pl.BlockSpec((bn, D), lambda i: (i, 0)),
            pl.BlockSpec((D, h1), lambda i: (0, 0)),
            pl.BlockSpec((1, h1), lambda i: (0, 0)),
            pl.BlockSpec((h1, h2), lambda i: (0, 0)),
            pl.BlockSpec((1, h2), lambda i: (0, 0)),
            pl.BlockSpec((h2, NTOPIC), lambda i: (0, 0)),
            pl.BlockSpec((1, NTOPIC), lambda i: (0, 0)),
        ],
        out_specs=pl.BlockSpec((bn, NTOPIC), lambda i: (i, 0)),
        out_shape=jax.ShapeDtypeStruct((n, NTOPIC), jnp.float32),
    )(x, w1, b1.reshape(1, -1), w2, b2.reshape(1, -1), w3, b3.reshape(1, -1))


SC_NCORE = 2
SC_NSUB = 16
EB = 128
K_GRP = 4
ZROWS = 64

_SC_ETS = (
    ("g2p", 10000, 50000, 50048, 51200, 163840),
    ("g2pr", 10000, 2000, 2048, 2048, 40960),
    ("p2pr", 50000, 2000, 2048, 2048, 81920),
    ("pr2pr", 2000, 2000, 2048, 2048, 24576),
)
ACC_ROWS = 51200


def _sc_agg_body(with_counts, *refs):
    n_out = 8 if with_counts else 4
    ins = refs[:12]
    outs = refs[12:12 + n_out]
    if with_counts:
        acc, idx_s, idx_d, rows, zbuf, ones, gsem, ssem = refs[12 + n_out:]
    else:
        acc, idx_s, idx_d, rows, zbuf, gsem, ssem = refs[12 + n_out:]
    cid = lax.axis_index("c")
    sid = lax.axis_index("s")

    def initz(i, carry):
        for j in range(CW // 16):
            zbuf[i, pl.ds(16 * j, 16)] = jnp.zeros((16,), jnp.float32)
        return carry

    lax.fori_loop(0, ZROWS, initz, 0)
    if with_counts:
        def inito(i, carry):
            for j in range(CW // 16):
                ones[i, pl.ds(16 * j, 16)] = jnp.ones((16,), jnp.float32)
            return carry

        lax.fori_loop(0, EB, inito, 0)

    for ei, (name, n_src, n_dst, nout, npad, epad) in enumerate(_SC_ETS):
        proj = ins[3 * ei]
        src2 = ins[3 * ei + 1]
        dst2 = ins[3 * ei + 2]
        out = outs[ei]
        ept = epad // SC_NSUB
        nb = ept // EB
        ngrp = nb // K_GRP
        rpt_zero = npad // SC_NSUB
        nzblk = rpt_zero // ZROWS
        wrt = nout // SC_NSUB

        def zero_acc(i, carry):
            pltpu.sync_copy(zbuf, acc.at[pl.ds(sid * rpt_zero + i * ZROWS, ZROWS)])
            return carry

        for c_l in range(NCHUNK // SC_NCORE):
            chunk = cid * (NCHUNK // SC_NCORE) + c_l
            lax.fori_loop(0, nzblk, zero_acc, 0)
            plsc.subcore_barrier()
            ptab = proj.at[chunk]

            def grp(g, carry):
                r0 = sid * nb + g * K_GRP
                pltpu.sync_copy(src2.at[pl.ds(r0, K_GRP)], idx_s)
                pltpu.sync_copy(dst2.at[pl.ds(r0, K_GRP)], idx_d)
                gds = [pltpu.async_copy(ptab.at[idx_s.at[k]], rows.at[k], gsem)
                       for k in range(K_GRP)]
                for d in gds:
                    d.wait()
                sds = [pltpu.async_copy(rows.at[k], acc.at[idx_d.at[k]], ssem,
                                        add=True)
                       for k in range(K_GRP)]
                for d in sds:
                    d.wait()
                return carry

            lax.fori_loop(0, ngrp, grp, 0)
            plsc.subcore_barrier()
            pltpu.sync_copy(acc.at[pl.ds(sid * wrt, wrt)],
                            out.at[chunk].at[pl.ds(sid * wrt, wrt)])
            plsc.subcore_barrier()

        if with_counts:
            cnt_out = outs[4 + ei]
            owner = 0 if ei < 2 else 1

            @pl.when(cid == owner)
            def _():
                lax.fori_loop(0, nzblk, zero_acc, 0)
                plsc.subcore_barrier()

                def cgrp(g, carry):
                    r0 = sid * nb + g * K_GRP
                    pltpu.sync_copy(dst2.at[pl.ds(r0, K_GRP)], idx_d)
                    sds = [pltpu.async_copy(ones, acc.at[idx_d.at[k]], ssem,
                                            add=True)
                           for k in range(K_GRP)]
                    for d in sds:
                        d.wait()
                    return carry

                lax.fori_loop(0, ngrp, cgrp, 0)
                plsc.subcore_barrier()
                pltpu.sync_copy(acc.at[pl.ds(sid * wrt, wrt)],
                                cnt_out.at[pl.ds(sid * wrt, wrt)])
                plsc.subcore_barrier()


def _sc_aggregate(projs, srcs2, dsts2, with_counts):
    out_type = [jax.ShapeDtypeStruct((NCHUNK, et[3], CW), jnp.float32)
                for et in _SC_ETS]
    if with_counts:
        out_type += [jax.ShapeDtypeStruct((et[3], CW), jnp.float32)
                     for et in _SC_ETS]
    scratch = [
        pltpu.VMEM_SHARED((ACC_ROWS, CW), jnp.float32),
        pltpu.VMEM((K_GRP, EB), jnp.int32),
        pltpu.VMEM((K_GRP, EB), jnp.int32),
        pltpu.VMEM((K_GRP, EB, CW), jnp.float32),
        pltpu.VMEM((ZROWS, CW), jnp.float32),
    ]
    if with_counts:
        scratch.append(pltpu.VMEM((EB, CW), jnp.float32))
    scratch += [pltpu.SemaphoreType.DMA, pltpu.SemaphoreType.DMA]
    mesh = plsc.VectorSubcoreMesh(core_axis_name="c", subcore_axis_name="s")
    fn = pl.kernel(
        functools.partial(_sc_agg_body, with_counts),
        out_type=out_type,
        mesh=mesh,
        scratch_types=scratch,
        compiler_params=pltpu.CompilerParams(use_tc_tiling_on_sc=False),
    )
    args = []
    for et in _SC_ETS:
        args += [projs[et[0]], srcs2[et[0]], dsts2[et[0]]]
    res = fn(*args)
    aggs = {et[0]: res[i] for i, et in enumerate(_SC_ETS)}
    cnts = None
    if with_counts:
        cnts = {et[0]: res[4 + i] for i, et in enumerate(_SC_ETS)}
    return aggs, cnts


def _pad_edges(edge, n_src, n_dst, npad, epad):
    e = edge.shape[1]
    extra = epad - e
    fill_src = jnp.arange(extra, dtype=jnp.int32) % n_src
    fill_dst = n_dst + jnp.arange(extra, dtype=jnp.int32) % (npad - n_dst)
    src = jnp.concatenate([edge[0], fill_src]).reshape(epad // EB, EB)
    dst = jnp.concatenate([edge[1], fill_dst]).reshape(epad // EB, EB)
    return src, dst


def kernel(gene_x, peak_x, protein_x, params, edge_g2p, edge_g2pr, edge_p2pr,
           edge_pr2pr):
    h = {"gene": gene_x, "peak": peak_x, "protein": protein_x}
    edges = {"g2p": edge_g2p, "g2pr": edge_g2pr, "p2pr": edge_p2pr,
             "pr2pr": edge_pr2pr}

    srcs2 = {}
    dsts2 = {}
    for name, n_src, n_dst, nout, npad, epad in _SC_ETS:
        srcs2[name], dsts2[name] = _pad_edges(edges[name], n_src, n_dst, npad,
                                              epad)

    cnts = None
    for l in range(NLAYERS):
        projs = {}
        selfs = {}
        for nt in NODE_TYPES:
            w_edges = [params["W_%s_%d" % (name, l)] for name in OUT_EDGES[nt]]
            outs = _project(h[nt], params["Wself_%s_%d" % (nt, l)], w_edges)
            selfs[nt] = outs[0]
            for j, name in enumerate(OUT_EDGES[nt]):
                projs[name] = outs[1 + j]

        aggs, new_cnts = _sc_aggregate(projs, srcs2, dsts2, cnts is None)
        if new_cnts is not None:
            cnts = new_cnts

        new_h = {}
        for nt in NODE_TYPES:
            names = IN_EDGES[nt]
            new_h[nt] = _combine(selfs[nt], h[nt],
                                 [aggs[m] for m in names],
                                 [cnts[m] for m in names])
        h = new_h

    outs = []
    for nt in NODE_TYPES:
        p = params
        outs.append(_encode(h[nt], p["enc_%s_W1" % nt], p["enc_%s_b1" % nt],
                            p["enc_%s_W2" % nt], p["enc_%s_b2" % nt],
                            p["enc_%s_W3" % nt], p["enc_%s_b3" % nt]))
    return tuple(outs)
